# baseline (speedup 1.0000x reference)
import functools
import math

import jax
import jax.numpy as jnp
from jax import lax
from jax.experimental import pallas as pl
from jax.experimental.pallas import tpu as pltpu
from jax.experimental.pallas import tpu_sc as plsc

LANES = 16
NBUF = 5
TC_BATCHES = 192


def _sc_lookup(text_batch, embed, pe2):
    B, L = text_batch.shape
    V, D = embed.shape
    scale = math.sqrt(D)

    info = plsc.get_sparse_core_info()
    NC, NS = info.num_cores, info.num_subcores
    NW = NC * NS
    BPW = B // NW
    PCH = 64
    NP = L // PCH
    NCH = BPW * NP

    mesh = plsc.VectorSubcoreMesh(core_axis_name="c", subcore_axis_name="s")

    @functools.partial(
        pl.kernel,
        mesh=mesh,
        out_type=jax.ShapeDtypeStruct((B, L, D), jnp.float32),
        scratch_types=(
            [pltpu.VMEM((L, D), jnp.float32),
             pltpu.VMEM((BPW * L,), jnp.int32)]
            + [pltpu.VMEM((PCH, D), jnp.float32) for _ in range(NBUF)]
            + [pltpu.SemaphoreType.DMA for _ in range(2 * NBUF)]
        ),
    )
    def emb_kernel(tb_hbm, emb_hbm, pe_hbm, out_hbm, pe_v, idx_v, *rest):
        g = rest[:NBUF]
        gsem = rest[NBUF:2 * NBUF]
        osem = rest[2 * NBUF:3 * NBUF]
        wid = lax.axis_index("s") * NC + lax.axis_index("c")

        pltpu.sync_copy(pe_hbm, pe_v)
        pltpu.sync_copy(tb_hbm.at[pl.ds(wid * (BPW * L), BPW * L)], idx_v)

        def gather_copy(i, slot):
            bl = i // NP
            p0 = (i % NP) * PCH
            return pltpu.make_async_copy(
                emb_hbm.at[idx_v.at[pl.ds(bl * L + p0, PCH)]],
                g[slot], gsem[slot])

        def out_copy(i, slot):
            bl = i // NP
            p0 = (i % NP) * PCH
            return pltpu.make_async_copy(
                g[slot],
                out_hbm.at[wid * BPW + bl, pl.ds(p0, PCH), :],
                osem[slot])

        def compute(i, slot):
            p0 = (i % NP) * PCH
            gb = g[slot]

            @plsc.parallel_loop(0, PCH, step=1, unroll=4)
            def _row(r):
                for kk in range(D // LANES):
                    sl = pl.ds(kk * LANES, LANES)
                    gb[r, sl] = gb[r, sl] * scale + pe_v[p0 + r, sl]

        def step(i, slot, fire_gather, wait_out):
            nslot = (slot + 3) % NBUF
            if wait_out:
                out_copy(i - 2, nslot).wait()
            if fire_gather:
                gather_copy(i + 3, nslot).start()
            gather_copy(i, slot).wait()
            compute(i, slot)
            out_copy(i, slot).start()

        for i in range(3):
            gather_copy(i, i).start()
        step(0, 0, fire_gather=True, wait_out=False)
        step(1, 1, fire_gather=True, wait_out=False)

        base = 2
        main = NCH - 3 - base
        trips = main // NBUF

        def trip_body(q, _):
            for j in range(NBUF):
                i = base + q * NBUF + j
                step(i, (base + j) % NBUF, fire_gather=True, wait_out=True)
            return 0

        lax.fori_loop(0, trips, trip_body, 0)
        for i in range(base + trips * NBUF, NCH - 3):
            step(i, i % NBUF, fire_gather=True, wait_out=True)

        for i in range(NCH - 3, NCH):
            step(i, i % NBUF, fire_gather=False, wait_out=True)
        out_copy(NCH - 2, (NCH - 2) % NBUF).wait()
        out_copy(NCH - 1, (NCH - 1) % NBUF).wait()

    return emb_kernel(text_batch.reshape(-1), embed, pe2)


def _tc_lookup(text_batch, embed, pe2):
    B, L = text_batch.shape
    V, D = embed.shape
    scale = math.sqrt(D)
    tb3 = text_batch.reshape(B, 1, L)

    def tc_body(idx_ref, table_ref, pe_ref, out_ref):
        def blk(t, _):
            rows = [table_ref[pl.ds(idx_ref[0, 0, t * 8 + j], 1), :]
                    for j in range(8)]
            r8 = jnp.concatenate(rows, axis=0)
            out_ref[0, pl.ds(t * 8, 8), :] = (
                r8 * scale + pe_ref[pl.ds(t * 8, 8), :])
            return 0

        lax.fori_loop(0, L // 8, blk, 0)

    return pl.pallas_call(
        tc_body,
        grid=(B,),
        in_specs=[
            pl.BlockSpec((1, 1, L), lambda b: (b, 0, 0),
                         memory_space=pltpu.SMEM),
            pl.BlockSpec(memory_space=pltpu.VMEM),
            pl.BlockSpec(memory_space=pltpu.VMEM),
        ],
        out_specs=pl.BlockSpec((1, L, D), lambda b: (b, 0, 0)),
        out_shape=jax.ShapeDtypeStruct((B, L, D), jnp.float32),
    )(tb3, embed, pe2)


def kernel(text_batch, embed, pe):
    B, L = text_batch.shape
    pe2 = pe.reshape(pe.shape[-2], pe.shape[-1])[:L]
    nsc = B - TC_BATCHES
    out_sc = _sc_lookup(text_batch[:nsc], embed, pe2)
    out_tc = (jnp.take(embed, text_batch[nsc:], axis=0) * math.sqrt(128.0)
              + pe2[None, :, :])
    return jnp.concatenate([out_sc, out_tc], axis=0)


# speedup vs baseline: 2.2645x; 2.2645x over previous
import functools
import math

import jax
import jax.numpy as jnp
from jax import lax
from jax.experimental import pallas as pl
from jax.experimental.pallas import tpu as pltpu
from jax.experimental.pallas import tpu_sc as plsc

LANES = 16
NBUF = 10
AHEAD = 5


def kernel(text_batch, embed, pe):
    B, L = text_batch.shape
    V, D = embed.shape
    scale = math.sqrt(D)
    pe2 = pe.reshape(pe.shape[-2], pe.shape[-1])[:L]

    info = plsc.get_sparse_core_info()
    NC, NS = info.num_cores, info.num_subcores
    NW = NC * NS
    BPW = B // NW
    PCH = 32
    NP = L // PCH
    NCH = BPW * NP

    mesh = plsc.VectorSubcoreMesh(core_axis_name="c", subcore_axis_name="s")

    @functools.partial(
        pl.kernel,
        mesh=mesh,
        out_type=jax.ShapeDtypeStruct((B, L, D), jnp.float32),
        scratch_types=(
            [pltpu.VMEM((L, D), jnp.float32),
             pltpu.VMEM((BPW, L), jnp.int32)]
            + [pltpu.VMEM((PCH, D), jnp.float32) for _ in range(NBUF)]
            + [pltpu.SemaphoreType.DMA for _ in range(2 * NBUF)]
        ),
    )
    def emb_kernel(tb_hbm, emb_hbm, pe_hbm, out_hbm, pe_v, idx_v, *rest):
        g = rest[:NBUF]
        gsem = rest[NBUF:2 * NBUF]
        osem = rest[2 * NBUF:3 * NBUF]
        wid = lax.axis_index("s") * NC + lax.axis_index("c")

        pltpu.sync_copy(pe_hbm, pe_v)
        pltpu.sync_copy(tb_hbm.at[pl.ds(wid * BPW, BPW), :], idx_v)

        def gather_copy(i, slot):
            bl = i // NP
            p0 = (i % NP) * PCH
            return pltpu.make_async_copy(
                emb_hbm.at[idx_v.at[bl, pl.ds(p0, PCH)]], g[slot], gsem[slot])

        def out_copy(i, slot):
            bl = i // NP
            p0 = (i % NP) * PCH
            return pltpu.make_async_copy(
                g[slot],
                out_hbm.at[wid * BPW + bl, pl.ds(p0, PCH), :],
                osem[slot])

        def compute(i, slot):
            p0 = (i % NP) * PCH
            gb = g[slot]

            @plsc.parallel_loop(0, PCH, step=1, unroll=4)
            def _row(r):
                for kk in range(D // LANES):
                    sl = pl.ds(kk * LANES, LANES)
                    gb[r, sl] = gb[r, sl] * scale + pe_v[p0 + r, sl]

        def step(i, slot, fire_gather, wait_out):
            nslot = (slot + AHEAD) % NBUF
            if wait_out:
                out_copy(i - (NBUF - AHEAD), nslot).wait()
            if fire_gather:
                gather_copy(i + AHEAD, nslot).start()
            gather_copy(i, slot).wait()
            compute(i, slot)
            out_copy(i, slot).start()

        for i in range(AHEAD):
            gather_copy(i, i % NBUF).start()
        for i in range(AHEAD):
            step(i, i % NBUF, fire_gather=True, wait_out=(i >= NBUF - AHEAD))

        base = AHEAD
        end = NCH - AHEAD
        trips = (end - base) // NBUF

        def trip_body(q, _):
            for j in range(NBUF):
                i = base + q * NBUF + j
                step(i, (base + j) % NBUF, fire_gather=True, wait_out=True)
            return 0

        lax.fori_loop(0, trips, trip_body, 0)
        for i in range(base + trips * NBUF, end):
            step(i, i % NBUF, fire_gather=True, wait_out=True)

        for i in range(end, NCH):
            step(i, i % NBUF, fire_gather=False, wait_out=True)
        for i in range(NCH - (NBUF - AHEAD), NCH):
            out_copy(i, i % NBUF).wait()

    return emb_kernel(text_batch, embed, pe2)


# speedup vs baseline: 2.7243x; 1.2030x over previous
import functools
import math

import jax
import jax.numpy as jnp
from jax import lax
from jax.experimental import pallas as pl
from jax.experimental.pallas import tpu as pltpu
from jax.experimental.pallas import tpu_sc as plsc

LANES = 16
NBUF = 5


def kernel(text_batch, embed, pe):
    B, L = text_batch.shape
    V, D = embed.shape
    scale = math.sqrt(D)
    pe2 = pe.reshape(pe.shape[-2], pe.shape[-1])[:L]

    info = plsc.get_sparse_core_info()
    NC, NS = info.num_cores, info.num_subcores
    NW = NC * NS
    BPW = B // NW
    PCH = 64
    NP = L // PCH
    NCH = BPW * NP

    mesh = plsc.VectorSubcoreMesh(core_axis_name="c", subcore_axis_name="s")

    @functools.partial(
        pl.kernel,
        mesh=mesh,
        out_type=jax.ShapeDtypeStruct((B, L, D), jnp.float32),
        scratch_types=(
            [pltpu.VMEM((L, D), jnp.float32),
             pltpu.VMEM((BPW, L), jnp.int32)]
            + [pltpu.VMEM((PCH, D), jnp.float32) for _ in range(NBUF)]
            + [pltpu.SemaphoreType.DMA for _ in range(2 * NBUF)]
        ),
    )
    def emb_kernel(tb_hbm, emb_hbm, pe_hbm, out_hbm, pe_v, idx_v, *rest):
        g = rest[:NBUF]
        gsem = rest[NBUF:2 * NBUF]
        osem = rest[2 * NBUF:3 * NBUF]
        wid = lax.axis_index("s") * NC + lax.axis_index("c")

        pltpu.sync_copy(pe_hbm, pe_v)
        pltpu.sync_copy(tb_hbm.at[pl.ds(wid * BPW, BPW), :], idx_v)

        def gather_copy(i, slot):
            bl = i // NP
            p0 = (i % NP) * PCH
            return pltpu.make_async_copy(
                emb_hbm.at[idx_v.at[bl, pl.ds(p0, PCH)]], g[slot], gsem[slot])

        def out_copy(i, slot):
            bl = i // NP
            p0 = (i % NP) * PCH
            return pltpu.make_async_copy(
                g[slot],
                out_hbm.at[wid * BPW + bl, pl.ds(p0, PCH), :],
                osem[slot])

        def compute(i, slot):
            p0 = (i % NP) * PCH
            gb = g[slot]

            @plsc.parallel_loop(0, PCH, step=1, unroll=8)
            def _row(r):
                for kk in range(D // LANES):
                    sl = pl.ds(kk * LANES, LANES)
                    gb[r, sl] = gb[r, sl] * scale + pe_v[p0 + r, sl]

        def step(i, slot, fire_gather, wait_out):
            nslot = (slot + 3) % NBUF
            if wait_out:
                out_copy(i - 2, nslot).wait()
            if fire_gather:
                gather_copy(i + 3, nslot).start()
            gather_copy(i, slot).wait()
            compute(i, slot)
            out_copy(i, slot).start()

        for i in range(3):
            gather_copy(i, i).start()
        step(0, 0, fire_gather=True, wait_out=False)
        step(1, 1, fire_gather=True, wait_out=False)

        base = 2
        main = NCH - 3 - base
        trips = main // NBUF

        def trip_body(q, _):
            for j in range(NBUF):
                i = base + q * NBUF + j
                step(i, (base + j) % NBUF, fire_gather=True, wait_out=True)
            return 0

        lax.fori_loop(0, trips, trip_body, 0)
        for i in range(base + trips * NBUF, NCH - 3):
            step(i, i % NBUF, fire_gather=True, wait_out=True)

        for i in range(NCH - 3, NCH):
            step(i, i % NBUF, fire_gather=False, wait_out=True)
        out_copy(NCH - 2, (NCH - 2) % NBUF).wait()
        out_copy(NCH - 1, (NCH - 1) % NBUF).wait()

    return emb_kernel(text_batch, embed, pe2)


# speedup vs baseline: 2.8137x; 1.0328x over previous
import functools
import math

import jax
import jax.numpy as jnp
from jax import lax
from jax.experimental import pallas as pl
from jax.experimental.pallas import tpu as pltpu
from jax.experimental.pallas import tpu_sc as plsc

LANES = 16
NBUF = 5
AHEAD = 3


def kernel(text_batch, embed, pe):
    B, L = text_batch.shape
    V, D = embed.shape
    scale = math.sqrt(D)
    pe2 = pe.reshape(pe.shape[-2], pe.shape[-1])[:L]

    info = plsc.get_sparse_core_info()
    NC, NS = info.num_cores, info.num_subcores
    NW = NC * NS
    PCH = 128
    NPS = L // PCH
    NBG = NW // NPS
    BPG = B // NBG
    NCH = BPG

    mesh = plsc.VectorSubcoreMesh(core_axis_name="c", subcore_axis_name="s")

    @functools.partial(
        pl.kernel,
        mesh=mesh,
        out_type=jax.ShapeDtypeStruct((B, L, D), jnp.float32),
        scratch_types=(
            [pltpu.VMEM((PCH, D), jnp.float32),
             pltpu.VMEM((BPG, PCH), jnp.int32)]
            + [pltpu.VMEM((PCH, D), jnp.float32) for _ in range(NBUF)]
            + [pltpu.SemaphoreType.DMA for _ in range(2 * NBUF)]
        ),
    )
    def emb_kernel(tb_hbm, emb_hbm, pe_hbm, out_hbm, pe_v, idx_v, *rest):
        g = rest[:NBUF]
        gsem = rest[NBUF:2 * NBUF]
        osem = rest[2 * NBUF:3 * NBUF]
        wid = lax.axis_index("s") * NC + lax.axis_index("c")
        bg = wid // NPS
        ps = wid % NPS
        b0 = bg * BPG
        p0 = ps * PCH

        pltpu.sync_copy(pe_hbm.at[pl.ds(p0, PCH), :], pe_v)
        pltpu.sync_copy(tb_hbm.at[pl.ds(b0, BPG), pl.ds(p0, PCH)], idx_v)

        def gather_copy(i, slot):
            return pltpu.make_async_copy(
                emb_hbm.at[idx_v.at[i]], g[slot], gsem[slot])

        def out_copy(i, slot):
            return pltpu.make_async_copy(
                g[slot], out_hbm.at[b0 + i, pl.ds(p0, PCH), :], osem[slot])

        def compute(slot):
            gb = g[slot]

            @plsc.parallel_loop(0, PCH, step=1, unroll=4)
            def _row(r):
                for kk in range(D // LANES):
                    sl = pl.ds(kk * LANES, LANES)
                    gb[r, sl] = gb[r, sl] * scale + pe_v[r, sl]

        def step(i, slot, fire_gather, wait_out):
            nslot = (slot + AHEAD) % NBUF
            if wait_out:
                out_copy(i - (NBUF - AHEAD), nslot).wait()
            if fire_gather:
                gather_copy(i + AHEAD, nslot).start()
            gather_copy(i, slot).wait()
            compute(slot)
            out_copy(i, slot).start()

        for i in range(AHEAD):
            gather_copy(i, i % NBUF).start()
        for i in range(AHEAD):
            step(i, i % NBUF, fire_gather=True, wait_out=(i >= NBUF - AHEAD))

        base = AHEAD
        end = NCH - AHEAD
        trips = (end - base) // NBUF

        def trip_body(q, _):
            for j in range(NBUF):
                i = base + q * NBUF + j
                step(i, (base + j) % NBUF, fire_gather=True, wait_out=True)
            return 0

        lax.fori_loop(0, trips, trip_body, 0)
        for i in range(base + trips * NBUF, end):
            step(i, i % NBUF, fire_gather=True, wait_out=True)

        for i in range(end, NCH):
            step(i, i % NBUF, fire_gather=False, wait_out=True)
        for i in range(NCH - (NBUF - AHEAD), NCH):
            out_copy(i, i % NBUF).wait()

    return emb_kernel(text_batch, embed, pe2)
